# baseline (speedup 1.0000x reference)
import jax
import jax.numpy as jnp
from jax.experimental import pallas as pl
from jax.experimental.pallas import tpu as pltpu

_EMBED = 32
_K = 1024
_N = 16384
_BLOCK = 512
_COMMIT = 0.25


def _stage(x, emb):
    x2 = jnp.sum(x ** 2, axis=1, keepdims=True)
    e2 = jnp.sum(emb ** 2, axis=1)
    mm = jax.lax.dot_general(x, emb, (((1,), (1,)), ((), ())))
    d = (x2 + e2) - 2.0 * mm
    m = jnp.min(d, axis=1, keepdims=True)
    col = jax.lax.broadcasted_iota(jnp.int32, d.shape, 1)
    idx = jnp.min(jnp.where(d == m, col, _K), axis=1)
    enc = (col == idx[:, None]).astype(jnp.float32)
    q = jax.lax.dot(enc, emb, precision=jax.lax.Precision.HIGHEST)
    return enc, q


def _vq_kernel(x_ref, e0_ref, e1_ref,
               q_ref, enc0_ref, enc1_ref, stats_ref,
               cnt_ref, acc_ref):
    i = pl.program_id(0)
    nsteps = pl.num_programs(0)

    @pl.when(i == 0)
    def _init():
        cnt_ref[...] = jnp.zeros_like(cnt_ref)
        acc_ref[...] = jnp.zeros_like(acc_ref)

    x = x_ref[...]
    enc0, q0 = _stage(x, e0_ref[...])
    r1 = x - q0
    enc1, q1 = _stage(r1, e1_ref[...])

    q_ref[...] = q0 + q1
    enc0_ref[...] = enc0
    enc1_ref[...] = enc1

    cnt_ref[0, :] += jnp.sum(enc0, axis=0)
    cnt_ref[1, :] += jnp.sum(enc1, axis=0)
    ssq = jnp.sum((q0 - x) ** 2) + jnp.sum((q1 - r1) ** 2)
    acc_ref[...] += ssq

    @pl.when(i == nsteps - 1)
    def _finish():
        loss_tile = acc_ref[...] * ((1.0 + _COMMIT) / (_N * _EMBED))
        p = cnt_ref[...] / _N
        s = jnp.sum(p * jnp.log(p + 1e-10), axis=1)
        ent = jnp.exp(-s)
        rid = jax.lax.broadcasted_iota(jnp.int32, ent.shape, 0)
        ent_tot = jnp.sum(jnp.where(rid < 2, ent, 0.0))
        row = jax.lax.broadcasted_iota(jnp.int32, stats_ref.shape, 0)
        colm = jax.lax.broadcasted_iota(jnp.int32, stats_ref.shape, 1)
        stats_ref[...] = jnp.where((row == 0) & (colm == 0), loss_tile,
                                   jnp.where((row == 0) & (colm == 1),
                                             ent_tot, 0.0))


def _run(flat, e0, e1, interpret=False):
    grid = (_N // _BLOCK,)
    q, enc0, enc1, stats = pl.pallas_call(
        _vq_kernel,
        grid=grid,
        in_specs=[
            pl.BlockSpec((_BLOCK, _EMBED), lambda i: (i, 0)),
            pl.BlockSpec((_K, _EMBED), lambda i: (0, 0)),
            pl.BlockSpec((_K, _EMBED), lambda i: (0, 0)),
        ],
        out_specs=[
            pl.BlockSpec((_BLOCK, _EMBED), lambda i: (i, 0)),
            pl.BlockSpec((_BLOCK, _K), lambda i: (i, 0)),
            pl.BlockSpec((_BLOCK, _K), lambda i: (i, 0)),
            pl.BlockSpec((8, 128), lambda i: (0, 0)),
        ],
        out_shape=[
            jax.ShapeDtypeStruct((_N, _EMBED), jnp.float32),
            jax.ShapeDtypeStruct((_N, _K), jnp.float32),
            jax.ShapeDtypeStruct((_N, _K), jnp.float32),
            jax.ShapeDtypeStruct((8, 128), jnp.float32),
        ],
        scratch_shapes=[
            pltpu.VMEM((8, _K), jnp.float32),
            pltpu.VMEM((8, 128), jnp.float32),
        ],
        compiler_params=pltpu.CompilerParams(
            dimension_semantics=("parallel",)),
        interpret=interpret,
    )(flat, e0, e1)
    return q, enc0, enc1, stats


def kernel(data, codebook0, codebook1):
    flat = data.reshape(-1, _EMBED)
    q, enc0, enc1, stats = _run(flat, codebook0, codebook1)
    quantized = q.reshape(data.shape)
    loss = stats[0, 0]
    entropy = stats[0, 1]
    return (quantized, (enc0, enc1), loss, entropy)


# speedup vs baseline: 1.0471x; 1.0471x over previous
import jax
import jax.numpy as jnp
from jax.experimental import pallas as pl
from jax.experimental.pallas import tpu as pltpu

_EMBED = 32
_K = 1024
_N = 16384
_BLOCK = 512
_SL = 128
_COMMIT = 0.25


def _stage(x, emb, enc_ref):
    ns = _K // _SL
    x2 = jnp.sum(x ** 2, axis=1, keepdims=True)
    e2 = jnp.sum(emb ** 2, axis=1)[None, :]
    mm = jax.lax.dot_general(x, emb, (((1,), (1,)), ((), ())))

    mval = None
    midx = None
    for s in range(ns):
        lo = s * _SL
        iota_s = jax.lax.broadcasted_iota(jnp.int32, (x.shape[0], _SL), 1) + lo
        ds = (x2 + e2[:, lo:lo + _SL]) - 2.0 * mm[:, lo:lo + _SL]
        if s == 0:
            mval, midx = ds, iota_s
        else:
            pred = ds < mval
            midx = jnp.where(pred, iota_s, midx)
            mval = jnp.minimum(mval, ds)
    m = jnp.min(mval, axis=1, keepdims=True)
    idx = jnp.min(jnp.where(mval == m, midx, _K), axis=1, keepdims=True)

    col = jax.lax.broadcasted_iota(jnp.int32, (x.shape[0], _K), 1)
    enc = (col == idx).astype(jnp.float32)
    enc_ref[...] = enc
    q = jax.lax.dot(enc, emb, precision=jax.lax.Precision.HIGHEST)
    return enc, q


def _vq_kernel(x_ref, e0_ref, e1_ref,
               q_ref, enc0_ref, enc1_ref, stats_ref,
               cnt_ref, acc_ref):
    i = pl.program_id(0)
    nsteps = pl.num_programs(0)

    @pl.when(i == 0)
    def _init():
        cnt_ref[...] = jnp.zeros_like(cnt_ref)
        acc_ref[...] = jnp.zeros_like(acc_ref)

    x = x_ref[...]
    enc0, q0 = _stage(x, e0_ref[...], enc0_ref)
    r1 = x - q0
    enc1, q1 = _stage(r1, e1_ref[...], enc1_ref)

    q_ref[...] = q0 + q1

    ones = jnp.ones((8, _BLOCK), jnp.float32)
    cnt_ref[0:8, :] += jax.lax.dot(ones, enc0)
    cnt_ref[8:16, :] += jax.lax.dot(ones, enc1)

    ssq = jnp.sum((q0 - x) ** 2) + jnp.sum((q1 - r1) ** 2)
    acc_ref[...] += ssq

    @pl.when(i == nsteps - 1)
    def _finish():
        loss_tile = acc_ref[...] * ((1.0 + _COMMIT) / (_N * _EMBED))
        p = cnt_ref[...] / _N
        s = jnp.sum(p * jnp.log(p + 1e-10), axis=1)
        ent = jnp.exp(-s)
        rid = jax.lax.broadcasted_iota(jnp.int32, ent.shape, 0)
        ent_tot = jnp.sum(jnp.where((rid == 0) | (rid == 8), ent, 0.0))
        row = jax.lax.broadcasted_iota(jnp.int32, stats_ref.shape, 0)
        colm = jax.lax.broadcasted_iota(jnp.int32, stats_ref.shape, 1)
        stats_ref[...] = jnp.where((row == 0) & (colm == 0), loss_tile,
                                   jnp.where((row == 0) & (colm == 1),
                                             ent_tot, 0.0))


def _run(flat, e0, e1, interpret=False):
    grid = (_N // _BLOCK,)
    q, enc0, enc1, stats = pl.pallas_call(
        _vq_kernel,
        grid=grid,
        in_specs=[
            pl.BlockSpec((_BLOCK, _EMBED), lambda i: (i, 0)),
            pl.BlockSpec((_K, _EMBED), lambda i: (0, 0)),
            pl.BlockSpec((_K, _EMBED), lambda i: (0, 0)),
        ],
        out_specs=[
            pl.BlockSpec((_BLOCK, _EMBED), lambda i: (i, 0)),
            pl.BlockSpec((_BLOCK, _K), lambda i: (i, 0)),
            pl.BlockSpec((_BLOCK, _K), lambda i: (i, 0)),
            pl.BlockSpec((8, 128), lambda i: (0, 0)),
        ],
        out_shape=[
            jax.ShapeDtypeStruct((_N, _EMBED), jnp.float32),
            jax.ShapeDtypeStruct((_N, _K), jnp.float32),
            jax.ShapeDtypeStruct((_N, _K), jnp.float32),
            jax.ShapeDtypeStruct((8, 128), jnp.float32),
        ],
        scratch_shapes=[
            pltpu.VMEM((16, _K), jnp.float32),
            pltpu.VMEM((8, 128), jnp.float32),
        ],
        interpret=interpret,
    )(flat, e0, e1)
    return q, enc0, enc1, stats


def kernel(data, codebook0, codebook1):
    flat = data.reshape(-1, _EMBED)
    q, enc0, enc1, stats = _run(flat, codebook0, codebook1)
    quantized = q.reshape(data.shape)
    loss = stats[0, 0]
    entropy = stats[0, 1]
    return (quantized, (enc0, enc1), loss, entropy)


# speedup vs baseline: 2.5601x; 2.4450x over previous
import jax
import jax.numpy as jnp
from jax.experimental import pallas as pl
from jax.experimental.pallas import tpu as pltpu

_EMBED = 32
_K = 1024
_N = 16384
_BLOCK = 512
_SL = 128
_COMMIT = 0.25


def _stage(x, emb, enc_ref):
    ns = _K // _SL
    x2 = jnp.sum(x ** 2, axis=1, keepdims=True)
    e2 = jnp.sum(emb ** 2, axis=1)[None, :]
    mm = jax.lax.dot_general(x, emb, (((1,), (1,)), ((), ())))

    mval = None
    midx = None
    for s in range(ns):
        lo = s * _SL
        iota_s = jax.lax.broadcasted_iota(jnp.int32, (x.shape[0], _SL), 1) + lo
        ds = (x2 + e2[:, lo:lo + _SL]) - 2.0 * mm[:, lo:lo + _SL]
        if s == 0:
            mval, midx = ds, iota_s
        else:
            pred = ds < mval
            midx = jnp.where(pred, iota_s, midx)
            mval = jnp.minimum(mval, ds)
    m = jnp.min(mval, axis=1, keepdims=True)
    idx = jnp.min(jnp.where(mval == m, midx, _K), axis=1, keepdims=True)

    col = jax.lax.broadcasted_iota(jnp.int32, (x.shape[0], _K), 1)
    enc = (col == idx).astype(jnp.float32)
    enc_ref[...] = enc
    q = jax.lax.dot(enc, emb)
    return enc, q


def _vq_kernel(x_ref, e0_ref, e1_ref,
               q_ref, enc0_ref, enc1_ref, stats_ref,
               cnt_ref, acc_ref):
    i = pl.program_id(0)
    nsteps = pl.num_programs(0)

    @pl.when(i == 0)
    def _init():
        cnt_ref[...] = jnp.zeros_like(cnt_ref)
        acc_ref[...] = jnp.zeros_like(acc_ref)

    x = x_ref[...]
    enc0, q0 = _stage(x, e0_ref[...], enc0_ref)
    r1 = x - q0
    enc1, q1 = _stage(r1, e1_ref[...], enc1_ref)

    q_ref[...] = q0 + q1

    ones = jnp.ones((8, _BLOCK), jnp.float32)
    cnt_ref[0:8, :] += jax.lax.dot(ones, enc0)
    cnt_ref[8:16, :] += jax.lax.dot(ones, enc1)

    ssq = jnp.sum((q0 - x) ** 2) + jnp.sum((q1 - r1) ** 2)
    acc_ref[...] += ssq

    @pl.when(i == nsteps - 1)
    def _finish():
        loss_tile = acc_ref[...] * ((1.0 + _COMMIT) / (_N * _EMBED))
        p = cnt_ref[...] / _N
        s = jnp.sum(p * jnp.log(p + 1e-10), axis=1)
        ent = jnp.exp(-s)
        rid = jax.lax.broadcasted_iota(jnp.int32, ent.shape, 0)
        ent_tot = jnp.sum(jnp.where((rid == 0) | (rid == 8), ent, 0.0))
        row = jax.lax.broadcasted_iota(jnp.int32, stats_ref.shape, 0)
        colm = jax.lax.broadcasted_iota(jnp.int32, stats_ref.shape, 1)
        stats_ref[...] = jnp.where((row == 0) & (colm == 0), loss_tile,
                                   jnp.where((row == 0) & (colm == 1),
                                             ent_tot, 0.0))


def _run(flat, e0, e1, interpret=False):
    grid = (_N // _BLOCK,)
    q, enc0, enc1, stats = pl.pallas_call(
        _vq_kernel,
        grid=grid,
        in_specs=[
            pl.BlockSpec((_BLOCK, _EMBED), lambda i: (i, 0)),
            pl.BlockSpec((_K, _EMBED), lambda i: (0, 0)),
            pl.BlockSpec((_K, _EMBED), lambda i: (0, 0)),
        ],
        out_specs=[
            pl.BlockSpec((_BLOCK, _EMBED), lambda i: (i, 0)),
            pl.BlockSpec((_BLOCK, _K), lambda i: (i, 0)),
            pl.BlockSpec((_BLOCK, _K), lambda i: (i, 0)),
            pl.BlockSpec((8, 128), lambda i: (0, 0)),
        ],
        out_shape=[
            jax.ShapeDtypeStruct((_N, _EMBED), jnp.float32),
            jax.ShapeDtypeStruct((_N, _K), jnp.float32),
            jax.ShapeDtypeStruct((_N, _K), jnp.float32),
            jax.ShapeDtypeStruct((8, 128), jnp.float32),
        ],
        scratch_shapes=[
            pltpu.VMEM((16, _K), jnp.float32),
            pltpu.VMEM((8, 128), jnp.float32),
        ],
        interpret=interpret,
    )(flat, e0, e1)
    return q, enc0, enc1, stats


def kernel(data, codebook0, codebook1):
    flat = data.reshape(-1, _EMBED)
    q, enc0, enc1, stats = _run(flat, codebook0, codebook1)
    quantized = q.reshape(data.shape)
    loss = stats[0, 0]
    entropy = stats[0, 1]
    return (quantized, (enc0, enc1), loss, entropy)
